# baseline (speedup 1.0000x reference)
import functools

import jax
import jax.numpy as jnp
from jax import lax
from jax.experimental import pallas as pl
from jax.experimental.pallas import tpu as pltpu
from jax.experimental.pallas import tpu_sc as plsc

NUM_EMB = 1000000
DIM = 64
NI = 16384
NJ = 20
NUM_CORES = 2
NUM_SUBCORES = 16
NW = NUM_CORES * NUM_SUBCORES
BLK = 128
NBI = NI // BLK
NBLK = NJ * NBI
BPW = NBLK // NW
NPAIR = BPW // 2


def _make_embedding_kernel():
    mesh = plsc.VectorSubcoreMesh(core_axis_name="c", subcore_axis_name="s")

    @functools.partial(
        pl.kernel,
        mesh=mesh,
        compiler_params=pltpu.CompilerParams(needs_layout_passes=False),
        out_type=jax.ShapeDtypeStruct((NJ, DIM, NI), jnp.float32),
        scratch_types=[
            pltpu.VMEM((2 * NI,), jnp.int32),
            pltpu.VMEM((BLK,), jnp.int32),
            pltpu.VMEM((BLK,), jnp.int32),
            pltpu.VMEM((BLK,), jnp.int32),
            pltpu.VMEM((BLK,), jnp.int32),
            pltpu.VMEM((BLK, 2 * DIM), jnp.float32),
            pltpu.VMEM((BLK, 2 * DIM), jnp.float32),
            pltpu.VMEM((DIM, BLK), jnp.float32),
            pltpu.VMEM((DIM, BLK), jnp.float32),
            pltpu.SemaphoreType.DMA,
            pltpu.SemaphoreType.DMA,
        ],
    )
    def emb(tok2d, w128, o3, idxb, m0, m1, h0, h1, r0, r1, o0, o1, sem_g, sem_s):
        wid = lax.axis_index("s") * NUM_CORES + lax.axis_index("c")
        base = wid * BPW
        j0 = lax.div(base, NBI)
        j1 = lax.div(base + BPW - 1, NBI)
        pltpu.sync_copy(tok2d.at[j0], idxb.at[pl.ds(0, NI)])
        pltpu.sync_copy(tok2d.at[j1], idxb.at[pl.ds(NI, NI)])

        ms = (m0, m1)
        hs = (h0, h1)
        rs = (r0, r1)
        os_ = (o0, o1)
        iotas = [lax.iota(jnp.int32, 16) + 16 * g for g in range(8)]

        def prep(blk, m_ref, h_ref):
            jj = lax.div(blk, NBI)
            ti = lax.rem(blk, NBI)
            off = (jj - j0) * NI + ti * BLK
            for g in range(8):
                v = idxb[pl.ds(off + 16 * g, 16)]
                m_ref[pl.ds(16 * g, 16)] = lax.shift_right_logical(v, 1)
                h_ref[pl.ds(16 * g, 16)] = lax.shift_left(v & 1, 6)

        def gather_start(m_ref, r_ref):
            pltpu.make_async_copy(w128.at[m_ref], r_ref, sem_g).start()

        def gather_wait(r_ref):
            pltpu.make_async_copy(w128.at[ms[0]], r_ref, sem_g).wait()

        def extract(r_ref, h_ref, o_ref):
            for g in range(8):
                hv = h_ref[pl.ds(16 * g, 16)]
                for cc in range(DIM):
                    x = plsc.load_gather(r_ref, [iotas[g], hv + cc])
                    o_ref[cc, pl.ds(16 * g, 16)] = x

        def store_start(blk, o_ref):
            jj = lax.div(blk, NBI)
            ti = lax.rem(blk, NBI)
            pltpu.make_async_copy(
                o_ref, o3.at[jj, :, pl.ds(ti * BLK, BLK)], sem_s
            ).start()

        def store_wait(o_ref):
            pltpu.make_async_copy(
                o_ref, o3.at[0, :, pl.ds(0, BLK)], sem_s
            ).wait()

        prep(base, m0, h0)
        gather_start(m0, r0)

        def pair_step(p, carry):
            for b in (0, 1):
                blk = base + 2 * p + b
                nxt = 1 - b

                @pl.when(2 * p + b < BPW - 1)
                def _():
                    prep(blk + 1, ms[nxt], hs[nxt])
                    gather_start(ms[nxt], rs[nxt])

                gather_wait(rs[b])

                @pl.when(2 * p + b >= 2)
                def _():
                    store_wait(os_[b])

                extract(rs[b], hs[b], os_[b])
                store_start(blk, os_[b])
            return carry

        lax.fori_loop(0, NPAIR, pair_step, 0)
        store_wait(o0)
        store_wait(o1)

    return emb


_emb = _make_embedding_kernel()


def kernel(token_ids, weight):
    tok2d = token_ids.T
    w128 = weight.reshape(NUM_EMB // 2, 2 * DIM)
    o3 = _emb(tok2d, w128)
    return o3.transpose(2, 0, 1)


# speedup vs baseline: 1.1216x; 1.1216x over previous
import functools

import jax
import jax.numpy as jnp
from jax import lax
from jax.experimental import pallas as pl
from jax.experimental.pallas import tpu as pltpu
from jax.experimental.pallas import tpu_sc as plsc

NUM_EMB = 1000000
DIM = 64
NI = 16384
NJ = 20
NUM_CORES = 2
NUM_SUBCORES = 16
NW = NUM_CORES * NUM_SUBCORES
BLK = 128
NBI = NI // BLK
NBLK = NJ * NBI
BPW = NBLK // NW
NPAIR = BPW // 2


def _make_embedding_kernel():
    mesh = plsc.VectorSubcoreMesh(core_axis_name="c", subcore_axis_name="s")

    @functools.partial(
        pl.kernel,
        mesh=mesh,
        compiler_params=pltpu.CompilerParams(needs_layout_passes=False),
        out_type=jax.ShapeDtypeStruct((NJ, DIM, NI), jnp.float32),
        scratch_types=[
            pltpu.VMEM((2 * NI,), jnp.int32),
            pltpu.VMEM((BLK,), jnp.int32),
            pltpu.VMEM((BLK,), jnp.int32),
            pltpu.VMEM((BLK,), jnp.int32),
            pltpu.VMEM((BLK,), jnp.int32),
            pltpu.VMEM((BLK, 2 * DIM), jnp.float32),
            pltpu.VMEM((BLK, 2 * DIM), jnp.float32),
            pltpu.VMEM((DIM, BLK), jnp.float32),
            pltpu.VMEM((DIM, BLK), jnp.float32),
            pltpu.SemaphoreType.DMA,
            pltpu.SemaphoreType.DMA,
        ],
    )
    def emb(tok2d, w128, o3, idxb, m0, m1, h0, h1, r0, r1, o0, o1, sem_g, sem_s):
        wid = lax.axis_index("s") * NUM_CORES + lax.axis_index("c")
        base = wid * BPW
        j0 = lax.div(base, NBI)
        j1 = lax.div(base + BPW - 1, NBI)
        pltpu.sync_copy(tok2d.at[j0], idxb.at[pl.ds(0, NI)])
        pltpu.sync_copy(tok2d.at[j1], idxb.at[pl.ds(NI, NI)])

        ms = (m0, m1)
        hs = (h0, h1)
        rs = (r0, r1)
        os_ = (o0, o1)
        iotas = [lax.iota(jnp.int32, 16) + 16 * g for g in range(8)]

        def prep(blk, m_ref, h_ref):
            jj = lax.div(blk, NBI)
            ti = lax.rem(blk, NBI)
            off = (jj - j0) * NI + ti * BLK
            for g in range(8):
                v = idxb[pl.ds(off + 16 * g, 16)]
                m_ref[pl.ds(16 * g, 16)] = lax.shift_right_logical(v, 1)
                h_ref[pl.ds(16 * g, 16)] = lax.shift_left(v & 1, 6)

        def gather_start(m_ref, r_ref):
            pltpu.make_async_copy(w128.at[m_ref], r_ref, sem_g).start()

        def gather_wait(r_ref):
            pltpu.make_async_copy(w128.at[ms[0]], r_ref, sem_g).wait()

        def extract(r_ref, h_ref, o_ref):
            hvs = [h_ref[pl.ds(16 * g, 16)] for g in range(8)]
            for cc in range(DIM):
                xs = [
                    plsc.load_gather(r_ref, [iotas[g], hvs[g] + cc])
                    for g in range(8)
                ]
                for g in range(8):
                    o_ref[cc, pl.ds(16 * g, 16)] = xs[g]

        def store_start(blk, o_ref):
            jj = lax.div(blk, NBI)
            ti = lax.rem(blk, NBI)
            pltpu.make_async_copy(
                o_ref, o3.at[jj, :, pl.ds(ti * BLK, BLK)], sem_s
            ).start()

        def store_wait(o_ref):
            pltpu.make_async_copy(
                o_ref, o3.at[0, :, pl.ds(0, BLK)], sem_s
            ).wait()

        prep(base, m0, h0)
        gather_start(m0, r0)

        def pair_step(p, carry):
            for b in (0, 1):
                blk = base + 2 * p + b
                nxt = 1 - b

                @pl.when(2 * p + b < BPW - 1)
                def _():
                    prep(blk + 1, ms[nxt], hs[nxt])
                    gather_start(ms[nxt], rs[nxt])

                gather_wait(rs[b])

                @pl.when(2 * p + b >= 2)
                def _():
                    store_wait(os_[b])

                extract(rs[b], hs[b], os_[b])
                store_start(blk, os_[b])
            return carry

        lax.fori_loop(0, NPAIR, pair_step, 0)
        store_wait(o0)
        store_wait(o1)

    return emb


_emb = _make_embedding_kernel()


def kernel(token_ids, weight):
    tok2d = token_ids.T
    w128 = weight.reshape(NUM_EMB // 2, 2 * DIM)
    o3 = _emb(tok2d, w128)
    return o3.transpose(2, 0, 1)
